# Initial kernel scaffold; baseline (speedup 1.0000x reference)
import jax
import jax.numpy as jnp
from jax.experimental import pallas as pl


def kernel(x, W_enc, b_enc, W_dec, b_dec):
    raise NotImplementedError("write your pallas kernel here")


# speedup vs baseline: 1.0004x; 1.0004x over previous
import functools

import jax
import jax.numpy as jnp
from jax.experimental import pallas as pl
from jax.experimental.pallas import tpu as pltpu

D_MODEL = 768
D_SPARSE = 24576
TOPK = 64
N_TOKENS = 1024

ENC_BN = 2048
DEC_BK = 2048


def _encode_body(x_ref, w_ref, b_ref, out_ref):
    acc = jax.lax.dot_general(
        x_ref[...], w_ref[...],
        dimension_numbers=(((1,), (1,)), ((), ())),
        preferred_element_type=jnp.float32,
    )
    out_ref[...] = jnp.maximum(acc + b_ref[...], 0.0)


def _encode(x_cent, W_enc, b_enc):
    grid = (D_SPARSE // ENC_BN,)
    return pl.pallas_call(
        _encode_body,
        grid=grid,
        in_specs=[
            pl.BlockSpec((N_TOKENS, D_MODEL), lambda n: (0, 0)),
            pl.BlockSpec((ENC_BN, D_MODEL), lambda n: (n, 0)),
            pl.BlockSpec((1, ENC_BN), lambda n: (0, n)),
        ],
        out_specs=pl.BlockSpec((N_TOKENS, ENC_BN), lambda n: (0, n)),
        out_shape=jax.ShapeDtypeStruct((N_TOKENS, D_SPARSE), jnp.float32),
    )(x_cent, W_enc, b_enc.reshape(1, D_SPARSE))


def _decode_body(z_ref, w_ref, b_ref, out_ref):
    k = pl.program_id(0)

    @pl.when(k == 0)
    def _init():
        out_ref[...] = jnp.broadcast_to(b_ref[...], out_ref.shape)

    out_ref[...] += jax.lax.dot_general(
        z_ref[...], w_ref[...],
        dimension_numbers=(((1,), (1,)), ((), ())),
        preferred_element_type=jnp.float32,
    )


def _decode(z, W_dec, b_dec):
    grid = (D_SPARSE // DEC_BK,)
    return pl.pallas_call(
        _decode_body,
        grid=grid,
        in_specs=[
            pl.BlockSpec((N_TOKENS, DEC_BK), lambda k: (0, k)),
            pl.BlockSpec((D_MODEL, DEC_BK), lambda k: (0, k)),
            pl.BlockSpec((1, D_MODEL), lambda k: (0, 0)),
        ],
        out_specs=pl.BlockSpec((N_TOKENS, D_MODEL), lambda k: (0, 0)),
        out_shape=jax.ShapeDtypeStruct((N_TOKENS, D_MODEL), jnp.float32),
    )(z, W_dec, b_dec.reshape(1, D_MODEL))


def kernel(x, W_enc, b_enc, W_dec, b_dec):
    x_cent = x - b_dec
    f = _encode(x_cent, W_enc, b_enc)
    topv, topi = jax.lax.top_k(f, TOPK)
    rows = jnp.arange(N_TOKENS)[:, None]
    z = jnp.zeros_like(f).at[rows, topi].set(topv)
    x_hat = _decode(z, W_dec, b_dec)
    return (x_hat, z)


# speedup vs baseline: 4.0249x; 4.0234x over previous
import functools

import jax
import jax.numpy as jnp
from jax import lax
from jax.experimental import pallas as pl
from jax.experimental.pallas import tpu as pltpu
from jax.experimental.pallas import tpu_sc as plsc

D_MODEL = 768
D_SPARSE = 24576
TOPK = 64
N_TOKENS = 1024

ENC_BN = 2048
DEC_BK = 2048

NC, NS, L = 2, 16, 16
NW = NC * NS
GROUPS = N_TOKENS // L
GPW = GROUPS // NW
CH = 512
NCHUNK = D_SPARSE // CH
NB1 = 2048
CAP = 1024


def _encode_body(x_ref, w_ref, b_ref, out_ref):
    acc = jax.lax.dot_general(
        x_ref[...], w_ref[...],
        dimension_numbers=(((1,), (1,)), ((), ())),
        preferred_element_type=jnp.float32,
    )
    out_ref[...] = jnp.maximum(acc + b_ref[...], 0.0) + 0.0


def _encode(x_cent, W_enc, b_enc):
    grid = (D_SPARSE // ENC_BN,)
    return pl.pallas_call(
        _encode_body,
        grid=grid,
        in_specs=[
            pl.BlockSpec((N_TOKENS, D_MODEL), lambda n: (0, 0)),
            pl.BlockSpec((ENC_BN, D_MODEL), lambda n: (n, 0)),
            pl.BlockSpec((1, ENC_BN), lambda n: (0, n)),
        ],
        out_specs=pl.BlockSpec((N_TOKENS, ENC_BN), lambda n: (0, n)),
        out_shape=jax.ShapeDtypeStruct((N_TOKENS, D_SPARSE), jnp.float32),
    )(x_cent, W_enc, b_enc.reshape(1, D_SPARSE))


def _decode_body(z_ref, w_ref, b_ref, out_ref):
    k = pl.program_id(0)

    @pl.when(k == 0)
    def _init():
        out_ref[...] = jnp.broadcast_to(b_ref[...], out_ref.shape)

    out_ref[...] += jax.lax.dot_general(
        z_ref[...], w_ref[...],
        dimension_numbers=(((1,), (1,)), ((), ())),
        preferred_element_type=jnp.float32,
    )


def _decode(z, W_dec, b_dec):
    grid = (D_SPARSE // DEC_BK,)
    return pl.pallas_call(
        _decode_body,
        grid=grid,
        in_specs=[
            pl.BlockSpec((N_TOKENS, DEC_BK), lambda k: (0, k)),
            pl.BlockSpec((D_MODEL, DEC_BK), lambda k: (0, k)),
            pl.BlockSpec((1, D_MODEL), lambda k: (0, 0)),
        ],
        out_specs=pl.BlockSpec((N_TOKENS, D_MODEL), lambda k: (0, 0)),
        out_shape=jax.ShapeDtypeStruct((N_TOKENS, D_MODEL), jnp.float32),
    )(z, W_dec, b_dec.reshape(1, D_MODEL))


def _lane():
    return lax.broadcasted_iota(jnp.int32, (L,), 0)


def _suffix_find(hist_ref, nbins, needed):
    def body(i, carry):
        acc, bsel, above, found = carry
        b = nbins - 1 - i
        c = hist_ref[pl.ds(b * L, L)]
        acc2 = acc + c
        crossed = jnp.logical_and(jnp.logical_not(found), acc2 >= needed)
        bsel = jnp.where(crossed, b, bsel)
        above = jnp.where(crossed, acc, above)
        found = jnp.logical_or(found, crossed)
        return acc2, bsel, above, found

    zeros = jnp.zeros((L,), jnp.int32)
    _, bsel, above, _ = lax.fori_loop(
        0, nbins, body, (zeros, zeros, zeros, jnp.zeros((L,), jnp.bool_)))
    return bsel, above


def _sc_body(f3, z, hist, h2, buf0, buf1, tbuf, cval, cidx, oval, oidx, zbuf,
             sem0, sem1):
    wid = lax.axis_index("s") * NC + lax.axis_index("c")
    lane = _lane()
    ones = jnp.ones((L,), jnp.int32)
    zeros_f = jnp.zeros((L,), jnp.float32)

    def zb(i, _):
        zbuf[pl.ds(i * L, L)] = zeros_f
        return 0
    lax.fori_loop(0, D_SPARSE // L, zb, 0)

    def do_group(g2, _):
        g = wid * GPW + g2

        def hz(i, _):
            hist[pl.ds(i * L, L)] = jnp.zeros((L,), jnp.int32)
            return 0
        lax.fori_loop(0, NB1, hz, 0)

        def oz(i, _):
            oval[pl.ds(i * L, L)] = jnp.zeros((L,), jnp.float32)
            oidx[pl.ds(i * L, L)] = jnp.zeros((L,), jnp.int32)
            return 0
        lax.fori_loop(0, TOPK, oz, 0)

        def stream_pass(process, carry0):
            pltpu.make_async_copy(
                f3.at[g, :, pl.ds(0 * CH, CH)], buf0, sem0).start()
            pltpu.make_async_copy(
                f3.at[g, :, pl.ds(1 * CH, CH)], buf1, sem1).start()

            def chunk(buf, sem, c, carry):
                pltpu.make_async_copy(
                    f3.at[g, :, pl.ds(c * CH, CH)], buf, sem).wait()

                def jb_body(jb, carry):
                    for t in range(L):
                        v = buf[t, pl.ds(jb * L, L)]
                        plsc.store_scatter(tbuf, [lane * L + t], v)
                    for q in range(L):
                        w = tbuf[pl.ds(q * L, L)]
                        carry = process(w, c * CH + jb * L + q, carry)
                    return carry

                carry = lax.fori_loop(0, CH // L, jb_body, carry)

                @pl.when(c + 2 < NCHUNK)
                def _():
                    pltpu.make_async_copy(
                        f3.at[g, :, pl.ds((c + 2) * CH, CH)], buf, sem
                    ).start()

                return carry

            def body(i, carry):
                carry = chunk(buf0, sem0, 2 * i, carry)
                carry = chunk(buf1, sem1, 2 * i + 1, carry)
                return carry

            return lax.fori_loop(0, NCHUNK // 2, body, carry0)

        def hist_elem(w, gi, carry):
            u = lax.bitcast_convert_type(w, jnp.uint32)
            d = lax.convert_element_type(
                lax.shift_right_logical(u, jnp.uint32(21)), jnp.int32)
            plsc.addupdate_scatter(hist, [d * L + lane], ones)
            return carry

        stream_pass(hist_elem, 0)

        needed64 = jnp.full((L,), TOPK, jnp.int32)
        bsel, above1 = _suffix_find(hist, NB1, needed64)

        def compact_elem(w, gi, cnt):
            u = lax.bitcast_convert_type(w, jnp.uint32)
            d = lax.convert_element_type(
                lax.shift_right_logical(u, jnp.uint32(21)), jnp.int32)
            m = jnp.logical_and(d >= bsel, cnt < CAP)
            addr = cnt * L + lane
            plsc.store_scatter(cval, [addr], w, mask=m)
            plsc.store_scatter(cidx, [addr], jnp.full((L,), 0, jnp.int32) + gi,
                               mask=m)
            return cnt + m.astype(jnp.int32)

        ncand = stream_pass(compact_elem, jnp.zeros((L,), jnp.int32))
        maxc = CAP

        needed = needed64 - above1
        tpfx = bsel

        def refine(tpfx, needed, hi_shift, shift, nbins):
            def hz2(i, _):
                h2[pl.ds(i * L, L)] = jnp.zeros((L,), jnp.int32)
                return 0
            lax.fori_loop(0, nbins, hz2, 0)

            def hbody(p, _):
                v = cval[pl.ds(p * L, L)]
                u = lax.bitcast_convert_type(v, jnp.uint32)
                valid = p < ncand
                hi = lax.convert_element_type(
                    lax.shift_right_logical(u, jnp.uint32(hi_shift)),
                    jnp.int32)
                match = jnp.logical_and(hi == tpfx, valid)
                d = lax.convert_element_type(
                    lax.shift_right_logical(u, jnp.uint32(shift)), jnp.int32
                ) & (nbins - 1)
                plsc.addupdate_scatter(h2, [d * L + lane], ones, mask=match)
                return 0
            lax.fori_loop(0, maxc, hbody, 0)

            b2, above = _suffix_find(h2, nbins, needed)
            return (tpfx * nbins + b2), needed - above

        tpfx, needed = refine(tpfx, needed, 21, 13, 256)
        tpfx, needed = refine(tpfx, needed, 13, 5, 256)
        tpfx, needed = refine(tpfx, needed, 5, 0, 32)
        tbits = lax.convert_element_type(tpfx, jnp.uint32)

        def sel_body(p, carry):
            cnt_out, cnt_eq = carry
            v = cval[pl.ds(p * L, L)]
            iv = cidx[pl.ds(p * L, L)]
            u = lax.bitcast_convert_type(v, jnp.uint32)
            valid = p < ncand
            m_gt = jnp.logical_and(u > tbits, valid)
            m_eq = jnp.logical_and(
                jnp.logical_and(u == tbits, valid), cnt_eq < needed)
            m = jnp.logical_and(jnp.logical_or(m_gt, m_eq), cnt_out < TOPK)
            addr = cnt_out * L + lane
            plsc.store_scatter(oval, [addr], v, mask=m)
            plsc.store_scatter(oidx, [addr], iv, mask=m)
            return (cnt_out + m.astype(jnp.int32),
                    cnt_eq + m_eq.astype(jnp.int32))

        lax.fori_loop(0, maxc, sel_body,
                      (jnp.zeros((L,), jnp.int32), jnp.zeros((L,), jnp.int32)))

        def zrow(r, _):
            for kk in range(TOPK // L):
                addr = (kk * L + lane) * L + r
                vals = plsc.load_gather(oval, [addr])
                idxs = plsc.load_gather(oidx, [addr])
                idxs = jnp.clip(idxs, 0, D_SPARSE - 1)
                plsc.store_scatter(zbuf, [idxs], vals)
            tok = g * L + r
            pltpu.sync_copy(zbuf, z.at[pl.ds(tok * D_SPARSE, D_SPARSE)])
            for kk in range(TOPK // L):
                addr = (kk * L + lane) * L + r
                idxs = plsc.load_gather(oidx, [addr])
                idxs = jnp.clip(idxs, 0, D_SPARSE - 1)
                plsc.store_scatter(zbuf, [idxs], zeros_f)
            return 0

        lax.fori_loop(0, L, zrow, 0)
        return 0

    lax.fori_loop(0, GPW, do_group, 0)


def _sc_topk_z(f3):
    mesh = plsc.VectorSubcoreMesh(core_axis_name="c", subcore_axis_name="s",
                                  num_cores=NC, num_subcores=NS)
    return pl.kernel(
        _sc_body,
        out_type=jax.ShapeDtypeStruct((N_TOKENS * D_SPARSE,), jnp.float32),
        mesh=mesh,
        compiler_params=pltpu.CompilerParams(needs_layout_passes=False),
        scratch_types=[
            pltpu.VMEM((NB1 * L,), jnp.int32),
            pltpu.VMEM((256 * L,), jnp.int32),
            pltpu.VMEM((L, CH), jnp.float32),
            pltpu.VMEM((L, CH), jnp.float32),
            pltpu.VMEM((L * L,), jnp.float32),
            pltpu.VMEM((CAP * L,), jnp.float32),
            pltpu.VMEM((CAP * L,), jnp.int32),
            pltpu.VMEM((TOPK * L,), jnp.float32),
            pltpu.VMEM((TOPK * L,), jnp.int32),
            pltpu.VMEM((D_SPARSE,), jnp.float32),
            pltpu.SemaphoreType.DMA,
            pltpu.SemaphoreType.DMA,
        ],
    )(f3)


def kernel(x, W_enc, b_enc, W_dec, b_dec):
    x_cent = x - b_dec
    f = _encode(x_cent, W_enc, b_enc)
    f3 = f.reshape(GROUPS, L, D_SPARSE)
    z = _sc_topk_z(f3).reshape(N_TOKENS, D_SPARSE)
    x_hat = _decode(z, W_dec, b_dec)
    return (x_hat, z)


# speedup vs baseline: 4.1834x; 1.0394x over previous
import functools

import jax
import jax.numpy as jnp
from jax import lax
from jax.experimental import pallas as pl
from jax.experimental.pallas import tpu as pltpu
from jax.experimental.pallas import tpu_sc as plsc

D_MODEL = 768
D_SPARSE = 24576
TOPK = 64
N_TOKENS = 1024

ENC_BN = 2048
DEC_BK = 2048

NC, NS, L = 2, 16, 16
NW = NC * NS
GROUPS = N_TOKENS // L
GPW = GROUPS // NW
CH = 512
NCHUNK = D_SPARSE // CH
NB1 = 2048
CAP = 1024


def _encode_body(x_ref, w_ref, b_ref, out_ref):
    acc = jax.lax.dot_general(
        x_ref[...], w_ref[...],
        dimension_numbers=(((1,), (1,)), ((), ())),
        preferred_element_type=jnp.float32,
    )
    out_ref[...] = jnp.maximum(acc + b_ref[...], 0.0) + 0.0


def _encode(x_cent, W_enc, b_enc):
    grid = (D_SPARSE // ENC_BN,)
    return pl.pallas_call(
        _encode_body,
        grid=grid,
        in_specs=[
            pl.BlockSpec((N_TOKENS, D_MODEL), lambda n: (0, 0)),
            pl.BlockSpec((ENC_BN, D_MODEL), lambda n: (n, 0)),
            pl.BlockSpec((1, ENC_BN), lambda n: (0, n)),
        ],
        out_specs=pl.BlockSpec((N_TOKENS, ENC_BN), lambda n: (0, n)),
        out_shape=jax.ShapeDtypeStruct((N_TOKENS, D_SPARSE), jnp.float32),
    )(x_cent, W_enc, b_enc.reshape(1, D_SPARSE))


def _decode_body(z_ref, w_ref, b_ref, out_ref):
    k = pl.program_id(0)

    @pl.when(k == 0)
    def _init():
        out_ref[...] = jnp.broadcast_to(b_ref[...], out_ref.shape)

    out_ref[...] += jax.lax.dot_general(
        z_ref[...], w_ref[...],
        dimension_numbers=(((1,), (1,)), ((), ())),
        preferred_element_type=jnp.float32,
    )


def _decode(z, W_dec, b_dec):
    grid = (D_SPARSE // DEC_BK,)
    return pl.pallas_call(
        _decode_body,
        grid=grid,
        in_specs=[
            pl.BlockSpec((N_TOKENS, DEC_BK), lambda k: (0, k)),
            pl.BlockSpec((D_MODEL, DEC_BK), lambda k: (0, k)),
            pl.BlockSpec((1, D_MODEL), lambda k: (0, 0)),
        ],
        out_specs=pl.BlockSpec((N_TOKENS, D_MODEL), lambda k: (0, 0)),
        out_shape=jax.ShapeDtypeStruct((N_TOKENS, D_MODEL), jnp.float32),
    )(z, W_dec, b_dec.reshape(1, D_MODEL))


def _lane():
    return lax.broadcasted_iota(jnp.int32, (L,), 0)


def _suffix_find(hist_ref, nbins, needed):
    def body(i, carry):
        acc, bsel, above, found = carry
        b = nbins - 1 - i
        c = hist_ref[pl.ds(b * L, L)]
        acc2 = acc + c
        crossed = jnp.logical_and(jnp.logical_not(found), acc2 >= needed)
        bsel = jnp.where(crossed, b, bsel)
        above = jnp.where(crossed, acc, above)
        found = jnp.logical_or(found, crossed)
        return acc2, bsel, above, found

    zeros = jnp.zeros((L,), jnp.int32)
    _, bsel, above, _ = lax.fori_loop(
        0, nbins, body, (zeros, zeros, zeros, jnp.zeros((L,), jnp.bool_)))
    return bsel, above


def _sc_body(f3, z, hist, h2, buf0, buf1, tbuf, cval, cidx, oval, oidx, zbuf,
             sem0, sem1):
    wid = lax.axis_index("s") * NC + lax.axis_index("c")
    lane = _lane()
    ones = jnp.ones((L,), jnp.int32)
    zeros_f = jnp.zeros((L,), jnp.float32)

    def zb(i, _):
        zbuf[pl.ds(i * L, L)] = zeros_f
        return 0
    lax.fori_loop(0, D_SPARSE // L, zb, 0)

    def do_group(g2, _):
        g = wid * GPW + g2

        def hz(i, _):
            hist[pl.ds(i * L, L)] = jnp.zeros((L,), jnp.int32)
            return 0
        lax.fori_loop(0, NB1, hz, 0)

        def oz(i, _):
            oval[pl.ds(i * L, L)] = jnp.zeros((L,), jnp.float32)
            oidx[pl.ds(i * L, L)] = jnp.zeros((L,), jnp.int32)
            return 0
        lax.fori_loop(0, TOPK, oz, 0)

        def stream_pass(process, carry0):
            pltpu.make_async_copy(
                f3.at[g, :, pl.ds(0 * CH, CH)], buf0, sem0).start()
            pltpu.make_async_copy(
                f3.at[g, :, pl.ds(1 * CH, CH)], buf1, sem1).start()

            def chunk(buf, sem, c, carry):
                pltpu.make_async_copy(
                    f3.at[g, :, pl.ds(c * CH, CH)], buf, sem).wait()

                def jb_body(jb, carry):
                    for t in range(L):
                        v = buf[t, pl.ds(jb * L, L)]
                        plsc.store_scatter(tbuf, [lane * L + t], v)
                    for q in range(L):
                        w = tbuf[pl.ds(q * L, L)]
                        carry = process(w, c * CH + jb * L + q, carry)
                    return carry

                carry = lax.fori_loop(0, CH // L, jb_body, carry)

                @pl.when(c + 2 < NCHUNK)
                def _():
                    pltpu.make_async_copy(
                        f3.at[g, :, pl.ds((c + 2) * CH, CH)], buf, sem
                    ).start()

                return carry

            def body(i, carry):
                carry = chunk(buf0, sem0, 2 * i, carry)
                carry = chunk(buf1, sem1, 2 * i + 1, carry)
                return carry

            return lax.fori_loop(0, NCHUNK // 2, body, carry0)

        def hist_elem(w, gi, carry):
            u = lax.bitcast_convert_type(w, jnp.uint32)
            d = lax.convert_element_type(
                lax.shift_right_logical(u, jnp.uint32(21)), jnp.int32)
            plsc.addupdate_scatter(hist, [d * L + lane], ones)
            return carry

        stream_pass(hist_elem, 0)

        needed64 = jnp.full((L,), TOPK, jnp.int32)
        bsel, above1 = _suffix_find(hist, NB1, needed64)

        def compact_elem(w, gi, cnt):
            u = lax.bitcast_convert_type(w, jnp.uint32)
            d = lax.convert_element_type(
                lax.shift_right_logical(u, jnp.uint32(21)), jnp.int32)
            m = jnp.logical_and(d >= bsel, cnt < CAP)
            addr = cnt * L + lane
            plsc.store_scatter(cval, [addr], w, mask=m)
            plsc.store_scatter(cidx, [addr], jnp.full((L,), 0, jnp.int32) + gi,
                               mask=m)
            return cnt + m.astype(jnp.int32)

        ncand = stream_pass(compact_elem, jnp.zeros((L,), jnp.int32))
        maxc = jnp.minimum(lax.reduce_max(ncand, axes=(0,)), CAP)

        needed = needed64 - above1
        tpfx = bsel

        def refine(tpfx, needed, hi_shift, shift, nbins):
            def hz2(i, _):
                h2[pl.ds(i * L, L)] = jnp.zeros((L,), jnp.int32)
                return 0
            lax.fori_loop(0, nbins, hz2, 0)

            def hbody(p, _):
                v = cval[pl.ds(p * L, L)]
                u = lax.bitcast_convert_type(v, jnp.uint32)
                valid = p < ncand
                hi = lax.convert_element_type(
                    lax.shift_right_logical(u, jnp.uint32(hi_shift)),
                    jnp.int32)
                match = jnp.logical_and(hi == tpfx, valid)
                d = lax.convert_element_type(
                    lax.shift_right_logical(u, jnp.uint32(shift)), jnp.int32
                ) & (nbins - 1)
                plsc.addupdate_scatter(h2, [d * L + lane], ones, mask=match)
                return 0
            lax.fori_loop(0, maxc, hbody, 0)

            b2, above = _suffix_find(h2, nbins, needed)
            return (tpfx * nbins + b2), needed - above

        tpfx, needed = refine(tpfx, needed, 21, 13, 256)
        tpfx, needed = refine(tpfx, needed, 13, 5, 256)
        tpfx, needed = refine(tpfx, needed, 5, 0, 32)
        tbits = lax.convert_element_type(tpfx, jnp.uint32)

        def sel_body(p, carry):
            cnt_out, cnt_eq = carry
            v = cval[pl.ds(p * L, L)]
            iv = cidx[pl.ds(p * L, L)]
            u = lax.bitcast_convert_type(v, jnp.uint32)
            valid = p < ncand
            m_gt = jnp.logical_and(u > tbits, valid)
            m_eq = jnp.logical_and(
                jnp.logical_and(u == tbits, valid), cnt_eq < needed)
            m = jnp.logical_and(jnp.logical_or(m_gt, m_eq), cnt_out < TOPK)
            addr = cnt_out * L + lane
            plsc.store_scatter(oval, [addr], v, mask=m)
            plsc.store_scatter(oidx, [addr], iv, mask=m)
            return (cnt_out + m.astype(jnp.int32),
                    cnt_eq + m_eq.astype(jnp.int32))

        lax.fori_loop(0, maxc, sel_body,
                      (jnp.zeros((L,), jnp.int32), jnp.zeros((L,), jnp.int32)))

        def zrow(r, _):
            for kk in range(TOPK // L):
                addr = (kk * L + lane) * L + r
                vals = plsc.load_gather(oval, [addr])
                idxs = plsc.load_gather(oidx, [addr])
                idxs = jnp.clip(idxs, 0, D_SPARSE - 1)
                plsc.store_scatter(zbuf, [idxs], vals)
            tok = g * L + r
            pltpu.sync_copy(zbuf, z.at[pl.ds(tok * D_SPARSE, D_SPARSE)])
            for kk in range(TOPK // L):
                addr = (kk * L + lane) * L + r
                idxs = plsc.load_gather(oidx, [addr])
                idxs = jnp.clip(idxs, 0, D_SPARSE - 1)
                plsc.store_scatter(zbuf, [idxs], zeros_f)
            return 0

        lax.fori_loop(0, L, zrow, 0)
        return 0

    lax.fori_loop(0, GPW, do_group, 0)


def _sc_topk_z(f3):
    mesh = plsc.VectorSubcoreMesh(core_axis_name="c", subcore_axis_name="s",
                                  num_cores=NC, num_subcores=NS)
    return pl.kernel(
        _sc_body,
        out_type=jax.ShapeDtypeStruct((N_TOKENS * D_SPARSE,), jnp.float32),
        mesh=mesh,
        compiler_params=pltpu.CompilerParams(needs_layout_passes=False),
        scratch_types=[
            pltpu.VMEM((NB1 * L,), jnp.int32),
            pltpu.VMEM((256 * L,), jnp.int32),
            pltpu.VMEM((L, CH), jnp.float32),
            pltpu.VMEM((L, CH), jnp.float32),
            pltpu.VMEM((L * L,), jnp.float32),
            pltpu.VMEM((CAP * L,), jnp.float32),
            pltpu.VMEM((CAP * L,), jnp.int32),
            pltpu.VMEM((TOPK * L,), jnp.float32),
            pltpu.VMEM((TOPK * L,), jnp.int32),
            pltpu.VMEM((D_SPARSE,), jnp.float32),
            pltpu.SemaphoreType.DMA,
            pltpu.SemaphoreType.DMA,
        ],
    )(f3)


def kernel(x, W_enc, b_enc, W_dec, b_dec):
    x_cent = x - b_dec
    f = _encode(x_cent, W_enc, b_enc)
    f3 = f.reshape(GROUPS, L, D_SPARSE)
    z = _sc_topk_z(f3).reshape(N_TOKENS, D_SPARSE)
    x_hat = _decode(z, W_dec, b_dec)
    return (x_hat, z)


# speedup vs baseline: 4.1836x; 1.0001x over previous
import functools

import jax
import jax.numpy as jnp
from jax import lax
from jax.experimental import pallas as pl
from jax.experimental.pallas import tpu as pltpu
from jax.experimental.pallas import tpu_sc as plsc

D_MODEL = 768
D_SPARSE = 24576
TOPK = 64
N_TOKENS = 1024

ENC_BN = 2048
DEC_BK = 2048

NC, NS, L = 2, 16, 16
NW = NC * NS
GROUPS = N_TOKENS // L
GPW = GROUPS // NW
CH = 512
NCHUNK = D_SPARSE // CH
NB1 = 2048
CAP = 1024


def _encode_body(x_ref, w_ref, b_ref, out_ref):
    acc = jax.lax.dot_general(
        x_ref[...], w_ref[...],
        dimension_numbers=(((1,), (1,)), ((), ())),
        preferred_element_type=jnp.float32,
    )
    out_ref[...] = jnp.maximum(acc + b_ref[...], 0.0) + 0.0


def _encode(x_cent, W_enc, b_enc):
    grid = (D_SPARSE // ENC_BN,)
    return pl.pallas_call(
        _encode_body,
        grid=grid,
        in_specs=[
            pl.BlockSpec((N_TOKENS, D_MODEL), lambda n: (0, 0)),
            pl.BlockSpec((ENC_BN, D_MODEL), lambda n: (n, 0)),
            pl.BlockSpec((1, ENC_BN), lambda n: (0, n)),
        ],
        out_specs=pl.BlockSpec((N_TOKENS, ENC_BN), lambda n: (0, n)),
        out_shape=jax.ShapeDtypeStruct((N_TOKENS, D_SPARSE), jnp.float32),
    )(x_cent, W_enc, b_enc.reshape(1, D_SPARSE))


def _decode_body(z_ref, w_ref, b_ref, out_ref):
    k = pl.program_id(0)

    @pl.when(k == 0)
    def _init():
        out_ref[...] = jnp.broadcast_to(b_ref[...], out_ref.shape)

    out_ref[...] += jax.lax.dot_general(
        z_ref[...].astype(jnp.bfloat16), w_ref[...].astype(jnp.bfloat16),
        dimension_numbers=(((1,), (1,)), ((), ())),
        preferred_element_type=jnp.float32,
    )


def _decode(z, W_dec, b_dec):
    grid = (D_SPARSE // DEC_BK,)
    return pl.pallas_call(
        _decode_body,
        grid=grid,
        in_specs=[
            pl.BlockSpec((N_TOKENS, DEC_BK), lambda k: (0, k)),
            pl.BlockSpec((D_MODEL, DEC_BK), lambda k: (0, k)),
            pl.BlockSpec((1, D_MODEL), lambda k: (0, 0)),
        ],
        out_specs=pl.BlockSpec((N_TOKENS, D_MODEL), lambda k: (0, 0)),
        out_shape=jax.ShapeDtypeStruct((N_TOKENS, D_MODEL), jnp.float32),
    )(z, W_dec, b_dec.reshape(1, D_MODEL))


def _lane():
    return lax.broadcasted_iota(jnp.int32, (L,), 0)


def _suffix_find(hist_ref, nbins, needed):
    def body(i, carry):
        acc, bsel, above, found = carry
        b = nbins - 1 - i
        c = hist_ref[pl.ds(b * L, L)]
        acc2 = acc + c
        crossed = jnp.logical_and(jnp.logical_not(found), acc2 >= needed)
        bsel = jnp.where(crossed, b, bsel)
        above = jnp.where(crossed, acc, above)
        found = jnp.logical_or(found, crossed)
        return acc2, bsel, above, found

    zeros = jnp.zeros((L,), jnp.int32)
    _, bsel, above, _ = lax.fori_loop(
        0, nbins, body, (zeros, zeros, zeros, jnp.zeros((L,), jnp.bool_)))
    return bsel, above


def _sc_body(f3, z, hist, h2, buf0, buf1, tbuf, cval, cidx, oval, oidx, zbuf,
             sem0, sem1):
    wid = lax.axis_index("s") * NC + lax.axis_index("c")
    lane = _lane()
    ones = jnp.ones((L,), jnp.int32)
    zeros_f = jnp.zeros((L,), jnp.float32)

    def zb(i, _):
        zbuf[pl.ds(i * L, L)] = zeros_f
        return 0
    lax.fori_loop(0, D_SPARSE // L, zb, 0)

    def do_group(g2, _):
        g = wid * GPW + g2

        def hz(i, _):
            hist[pl.ds(i * L, L)] = jnp.zeros((L,), jnp.int32)
            return 0
        lax.fori_loop(0, NB1, hz, 0)

        def oz(i, _):
            oval[pl.ds(i * L, L)] = jnp.zeros((L,), jnp.float32)
            oidx[pl.ds(i * L, L)] = jnp.zeros((L,), jnp.int32)
            return 0
        lax.fori_loop(0, TOPK, oz, 0)

        def stream_pass(process, carry0):
            pltpu.make_async_copy(
                f3.at[g, :, pl.ds(0 * CH, CH)], buf0, sem0).start()
            pltpu.make_async_copy(
                f3.at[g, :, pl.ds(1 * CH, CH)], buf1, sem1).start()

            def chunk(buf, sem, c, carry):
                pltpu.make_async_copy(
                    f3.at[g, :, pl.ds(c * CH, CH)], buf, sem).wait()

                def jb_body(jb, carry):
                    for t in range(L):
                        v = buf[t, pl.ds(jb * L, L)]
                        plsc.store_scatter(tbuf, [lane * L + t], v)
                    for q in range(L):
                        w = tbuf[pl.ds(q * L, L)]
                        carry = process(w, c * CH + jb * L + q, carry)
                    return carry

                carry = lax.fori_loop(0, CH // L, jb_body, carry)

                @pl.when(c + 2 < NCHUNK)
                def _():
                    pltpu.make_async_copy(
                        f3.at[g, :, pl.ds((c + 2) * CH, CH)], buf, sem
                    ).start()

                return carry

            def body(i, carry):
                carry = chunk(buf0, sem0, 2 * i, carry)
                carry = chunk(buf1, sem1, 2 * i + 1, carry)
                return carry

            return lax.fori_loop(0, NCHUNK // 2, body, carry0)

        def hist_elem(w, gi, carry):
            u = lax.bitcast_convert_type(w, jnp.uint32)
            d = lax.convert_element_type(
                lax.shift_right_logical(u, jnp.uint32(21)), jnp.int32)
            plsc.addupdate_scatter(hist, [d * L + lane], ones)
            return carry

        stream_pass(hist_elem, 0)

        needed64 = jnp.full((L,), TOPK, jnp.int32)
        bsel, above1 = _suffix_find(hist, NB1, needed64)

        def compact_elem(w, gi, cnt):
            u = lax.bitcast_convert_type(w, jnp.uint32)
            d = lax.convert_element_type(
                lax.shift_right_logical(u, jnp.uint32(21)), jnp.int32)
            m = jnp.logical_and(d >= bsel, cnt < CAP)
            addr = cnt * L + lane
            plsc.store_scatter(cval, [addr], w, mask=m)
            plsc.store_scatter(cidx, [addr], jnp.full((L,), 0, jnp.int32) + gi,
                               mask=m)
            return cnt + m.astype(jnp.int32)

        ncand = stream_pass(compact_elem, jnp.zeros((L,), jnp.int32))
        maxc = jnp.minimum(lax.reduce_max(ncand, axes=(0,)), CAP)

        needed = needed64 - above1
        tpfx = bsel

        def refine(tpfx, needed, hi_shift, shift, nbins):
            def hz2(i, _):
                h2[pl.ds(i * L, L)] = jnp.zeros((L,), jnp.int32)
                return 0
            lax.fori_loop(0, nbins, hz2, 0)

            def hbody(p, _):
                v = cval[pl.ds(p * L, L)]
                u = lax.bitcast_convert_type(v, jnp.uint32)
                valid = p < ncand
                hi = lax.convert_element_type(
                    lax.shift_right_logical(u, jnp.uint32(hi_shift)),
                    jnp.int32)
                match = jnp.logical_and(hi == tpfx, valid)
                d = lax.convert_element_type(
                    lax.shift_right_logical(u, jnp.uint32(shift)), jnp.int32
                ) & (nbins - 1)
                plsc.addupdate_scatter(h2, [d * L + lane], ones, mask=match)
                return 0
            lax.fori_loop(0, maxc, hbody, 0)

            b2, above = _suffix_find(h2, nbins, needed)
            return (tpfx * nbins + b2), needed - above

        tpfx, needed = refine(tpfx, needed, 21, 13, 256)
        tpfx, needed = refine(tpfx, needed, 13, 5, 256)
        tpfx, needed = refine(tpfx, needed, 5, 0, 32)
        tbits = lax.convert_element_type(tpfx, jnp.uint32)

        def sel_body(p, carry):
            cnt_out, cnt_eq = carry
            v = cval[pl.ds(p * L, L)]
            iv = cidx[pl.ds(p * L, L)]
            u = lax.bitcast_convert_type(v, jnp.uint32)
            valid = p < ncand
            m_gt = jnp.logical_and(u > tbits, valid)
            m_eq = jnp.logical_and(
                jnp.logical_and(u == tbits, valid), cnt_eq < needed)
            m = jnp.logical_and(jnp.logical_or(m_gt, m_eq), cnt_out < TOPK)
            addr = cnt_out * L + lane
            plsc.store_scatter(oval, [addr], v, mask=m)
            plsc.store_scatter(oidx, [addr], iv, mask=m)
            return (cnt_out + m.astype(jnp.int32),
                    cnt_eq + m_eq.astype(jnp.int32))

        lax.fori_loop(0, maxc, sel_body,
                      (jnp.zeros((L,), jnp.int32), jnp.zeros((L,), jnp.int32)))

        def zrow(r, _):
            for kk in range(TOPK // L):
                addr = (kk * L + lane) * L + r
                vals = plsc.load_gather(oval, [addr])
                idxs = plsc.load_gather(oidx, [addr])
                idxs = jnp.clip(idxs, 0, D_SPARSE - 1)
                plsc.store_scatter(zbuf, [idxs], vals)
            tok = g * L + r
            pltpu.sync_copy(zbuf, z.at[pl.ds(tok * D_SPARSE, D_SPARSE)])
            for kk in range(TOPK // L):
                addr = (kk * L + lane) * L + r
                idxs = plsc.load_gather(oidx, [addr])
                idxs = jnp.clip(idxs, 0, D_SPARSE - 1)
                plsc.store_scatter(zbuf, [idxs], zeros_f)
            return 0

        lax.fori_loop(0, L, zrow, 0)
        return 0

    lax.fori_loop(0, GPW, do_group, 0)


def _sc_topk_z(f3):
    mesh = plsc.VectorSubcoreMesh(core_axis_name="c", subcore_axis_name="s",
                                  num_cores=NC, num_subcores=NS)
    return pl.kernel(
        _sc_body,
        out_type=jax.ShapeDtypeStruct((N_TOKENS * D_SPARSE,), jnp.float32),
        mesh=mesh,
        compiler_params=pltpu.CompilerParams(needs_layout_passes=False),
        scratch_types=[
            pltpu.VMEM((NB1 * L,), jnp.int32),
            pltpu.VMEM((256 * L,), jnp.int32),
            pltpu.VMEM((L, CH), jnp.float32),
            pltpu.VMEM((L, CH), jnp.float32),
            pltpu.VMEM((L * L,), jnp.float32),
            pltpu.VMEM((CAP * L,), jnp.float32),
            pltpu.VMEM((CAP * L,), jnp.int32),
            pltpu.VMEM((TOPK * L,), jnp.float32),
            pltpu.VMEM((TOPK * L,), jnp.int32),
            pltpu.VMEM((D_SPARSE,), jnp.float32),
            pltpu.SemaphoreType.DMA,
            pltpu.SemaphoreType.DMA,
        ],
    )(f3)


def kernel(x, W_enc, b_enc, W_dec, b_dec):
    x_cent = x - b_dec
    f = _encode(x_cent, W_enc, b_enc)
    f3 = f.reshape(GROUPS, L, D_SPARSE)
    z = _sc_topk_z(f3).reshape(N_TOKENS, D_SPARSE)
    x_hat = _decode(z, W_dec, b_dec)
    return (x_hat, z)


# speedup vs baseline: 4.1843x; 1.0002x over previous
import jax
import jax.numpy as jnp
from jax import lax
from jax.experimental import pallas as pl
from jax.experimental.pallas import tpu as pltpu
from jax.experimental.pallas import tpu_sc as plsc

D_MODEL = 768
D_SPARSE = 24576
TOPK = 64
N_TOKENS = 1024

ENC_BN = 2048
DEC_BK = 2048

NC, NS, L = 2, 16, 16
NW = NC * NS
GROUPS = N_TOKENS // L
GPW = GROUPS // NW
CH = 512
NCHUNK = D_SPARSE // CH
NB1 = 2048
CAP = 1024


def _encode_body(x_ref, w_ref, b_ref, out_ref):
    acc = jax.lax.dot_general(
        x_ref[...], w_ref[...],
        dimension_numbers=(((1,), (1,)), ((), ())),
        preferred_element_type=jnp.float32,
    )
    out_ref[...] = jnp.maximum(acc + b_ref[...], 0.0) + 0.0


def _encode(x_cent, W_enc, b_enc):
    grid = (D_SPARSE // ENC_BN,)
    return pl.pallas_call(
        _encode_body,
        grid=grid,
        in_specs=[
            pl.BlockSpec((N_TOKENS, D_MODEL), lambda n: (0, 0)),
            pl.BlockSpec((ENC_BN, D_MODEL), lambda n: (n, 0)),
            pl.BlockSpec((1, ENC_BN), lambda n: (0, n)),
        ],
        out_specs=pl.BlockSpec((N_TOKENS, ENC_BN), lambda n: (0, n)),
        out_shape=jax.ShapeDtypeStruct((N_TOKENS, D_SPARSE), jnp.float32),
    )(x_cent, W_enc, b_enc.reshape(1, D_SPARSE))


def _decode_body(z_ref, w_ref, b_ref, out_ref):
    k = pl.program_id(0)

    @pl.when(k == 0)
    def _init():
        out_ref[...] = jnp.broadcast_to(b_ref[...], out_ref.shape)

    out_ref[...] += jax.lax.dot_general(
        z_ref[...], w_ref[...],
        dimension_numbers=(((1,), (1,)), ((), ())),
        preferred_element_type=jnp.float32,
    )


def _decode(z, W_dec, b_dec):
    grid = (D_SPARSE // DEC_BK,)
    return pl.pallas_call(
        _decode_body,
        grid=grid,
        in_specs=[
            pl.BlockSpec((N_TOKENS, DEC_BK), lambda k: (0, k)),
            pl.BlockSpec((D_MODEL, DEC_BK), lambda k: (0, k)),
            pl.BlockSpec((1, D_MODEL), lambda k: (0, 0)),
        ],
        out_specs=pl.BlockSpec((N_TOKENS, D_MODEL), lambda k: (0, 0)),
        out_shape=jax.ShapeDtypeStruct((N_TOKENS, D_MODEL), jnp.float32),
    )(z, W_dec, b_dec.reshape(1, D_MODEL))


def _lane():
    return lax.broadcasted_iota(jnp.int32, (L,), 0)


def _suffix_find(hist_ref, nbins, needed):
    def body(i, carry):
        acc, bsel, above, found = carry
        b = nbins - 1 - i
        c = hist_ref[pl.ds(b * L, L)]
        acc2 = acc + c
        crossed = jnp.logical_and(jnp.logical_not(found), acc2 >= needed)
        bsel = jnp.where(crossed, b, bsel)
        above = jnp.where(crossed, acc, above)
        found = jnp.logical_or(found, crossed)
        return acc2, bsel, above, found

    zeros = jnp.zeros((L,), jnp.int32)
    _, bsel, above, _ = lax.fori_loop(
        0, nbins, body, (zeros, zeros, zeros, jnp.zeros((L,), jnp.bool_)))
    return bsel, above


def _sc_body(f3, z, hist, h2, buf0, buf1, tbuf, cval, cidx, oval, oidx, zbuf,
             sem0, sem1):
    wid = lax.axis_index("s") * NC + lax.axis_index("c")
    lane = _lane()
    ones = jnp.ones((L,), jnp.int32)
    zeros_f = jnp.zeros((L,), jnp.float32)

    def zb(i, _):
        zbuf[pl.ds(i * L, L)] = zeros_f
        return 0
    lax.fori_loop(0, D_SPARSE // L, zb, 0)

    def do_group(g2, _):
        g = wid * GPW + g2

        def hz(i, _):
            hist[pl.ds(i * L, L)] = jnp.zeros((L,), jnp.int32)
            return 0
        lax.fori_loop(0, NB1, hz, 0)

        def oz(i, _):
            oval[pl.ds(i * L, L)] = jnp.zeros((L,), jnp.float32)
            oidx[pl.ds(i * L, L)] = jnp.zeros((L,), jnp.int32)
            return 0
        lax.fori_loop(0, TOPK, oz, 0)

        def stream_pass(process, carry0):
            pltpu.make_async_copy(
                f3.at[g, :, pl.ds(0 * CH, CH)], buf0, sem0).start()
            pltpu.make_async_copy(
                f3.at[g, :, pl.ds(1 * CH, CH)], buf1, sem1).start()

            def chunk(buf, sem, c, carry):
                pltpu.make_async_copy(
                    f3.at[g, :, pl.ds(c * CH, CH)], buf, sem).wait()

                def jb_body(jb, carry):
                    for t in range(L):
                        v = buf[t, pl.ds(jb * L, L)]
                        plsc.store_scatter(tbuf, [lane * L + t], v)
                    for q in range(L):
                        w = tbuf[pl.ds(q * L, L)]
                        carry = process(w, c * CH + jb * L + q, carry)
                    return carry

                carry = lax.fori_loop(0, CH // L, jb_body, carry)

                @pl.when(c + 2 < NCHUNK)
                def _():
                    pltpu.make_async_copy(
                        f3.at[g, :, pl.ds((c + 2) * CH, CH)], buf, sem
                    ).start()

                return carry

            def body(i, carry):
                carry = chunk(buf0, sem0, 2 * i, carry)
                carry = chunk(buf1, sem1, 2 * i + 1, carry)
                return carry

            return lax.fori_loop(0, NCHUNK // 2, body, carry0)

        def hist_elem(w, gi, carry):
            u = lax.bitcast_convert_type(w, jnp.uint32)
            d = lax.convert_element_type(
                lax.shift_right_logical(u, jnp.uint32(21)), jnp.int32)
            plsc.addupdate_scatter(hist, [d * L + lane], ones)
            return carry

        stream_pass(hist_elem, 0)

        needed64 = jnp.full((L,), TOPK, jnp.int32)
        bsel, above1 = _suffix_find(hist, NB1, needed64)

        def compact_elem(w, gi, cnt):
            u = lax.bitcast_convert_type(w, jnp.uint32)
            d = lax.convert_element_type(
                lax.shift_right_logical(u, jnp.uint32(21)), jnp.int32)
            m = jnp.logical_and(d >= bsel, cnt < CAP)
            addr = cnt * L + lane
            plsc.store_scatter(cval, [addr], w, mask=m)
            plsc.store_scatter(cidx, [addr], jnp.full((L,), 0, jnp.int32) + gi,
                               mask=m)
            return cnt + m.astype(jnp.int32)

        ncand = stream_pass(compact_elem, jnp.zeros((L,), jnp.int32))
        maxc = jnp.minimum(lax.reduce_max(ncand, axes=(0,)), CAP)

        needed = needed64 - above1
        tpfx = bsel

        def refine(tpfx, needed, hi_shift, shift, nbins):
            def hz2(i, _):
                h2[pl.ds(i * L, L)] = jnp.zeros((L,), jnp.int32)
                return 0
            lax.fori_loop(0, nbins, hz2, 0)

            def hbody(p, _):
                v = cval[pl.ds(p * L, L)]
                u = lax.bitcast_convert_type(v, jnp.uint32)
                valid = p < ncand
                hi = lax.convert_element_type(
                    lax.shift_right_logical(u, jnp.uint32(hi_shift)),
                    jnp.int32)
                match = jnp.logical_and(hi == tpfx, valid)
                d = lax.convert_element_type(
                    lax.shift_right_logical(u, jnp.uint32(shift)), jnp.int32
                ) & (nbins - 1)
                plsc.addupdate_scatter(h2, [d * L + lane], ones, mask=match)
                return 0
            lax.fori_loop(0, maxc, hbody, 0)

            b2, above = _suffix_find(h2, nbins, needed)
            return (tpfx * nbins + b2), needed - above

        tpfx, needed = refine(tpfx, needed, 21, 13, 256)
        tpfx, needed = refine(tpfx, needed, 13, 5, 256)
        tpfx, needed = refine(tpfx, needed, 5, 0, 32)
        tbits = lax.convert_element_type(tpfx, jnp.uint32)

        def sel_body(p, carry):
            cnt_out, cnt_eq = carry
            v = cval[pl.ds(p * L, L)]
            iv = cidx[pl.ds(p * L, L)]
            u = lax.bitcast_convert_type(v, jnp.uint32)
            valid = p < ncand
            m_gt = jnp.logical_and(u > tbits, valid)
            m_eq = jnp.logical_and(
                jnp.logical_and(u == tbits, valid), cnt_eq < needed)
            m = jnp.logical_and(jnp.logical_or(m_gt, m_eq), cnt_out < TOPK)
            addr = cnt_out * L + lane
            plsc.store_scatter(oval, [addr], v, mask=m)
            plsc.store_scatter(oidx, [addr], iv, mask=m)
            return (cnt_out + m.astype(jnp.int32),
                    cnt_eq + m_eq.astype(jnp.int32))

        lax.fori_loop(0, maxc, sel_body,
                      (jnp.zeros((L,), jnp.int32), jnp.zeros((L,), jnp.int32)))

        def zrow(r, _):
            for kk in range(TOPK // L):
                addr = (kk * L + lane) * L + r
                vals = plsc.load_gather(oval, [addr])
                idxs = plsc.load_gather(oidx, [addr])
                idxs = jnp.clip(idxs, 0, D_SPARSE - 1)
                plsc.store_scatter(zbuf, [idxs], vals)
            tok = g * L + r
            pltpu.sync_copy(zbuf, z.at[pl.ds(tok * D_SPARSE, D_SPARSE)])
            for kk in range(TOPK // L):
                addr = (kk * L + lane) * L + r
                idxs = plsc.load_gather(oidx, [addr])
                idxs = jnp.clip(idxs, 0, D_SPARSE - 1)
                plsc.store_scatter(zbuf, [idxs], zeros_f)
            return 0

        lax.fori_loop(0, L, zrow, 0)
        return 0

    lax.fori_loop(0, GPW, do_group, 0)


def _sc_topk_z(f3):
    mesh = plsc.VectorSubcoreMesh(core_axis_name="c", subcore_axis_name="s",
                                  num_cores=NC, num_subcores=NS)
    return pl.kernel(
        _sc_body,
        out_type=jax.ShapeDtypeStruct((N_TOKENS * D_SPARSE,), jnp.float32),
        mesh=mesh,
        compiler_params=pltpu.CompilerParams(needs_layout_passes=False),
        scratch_types=[
            pltpu.VMEM((NB1 * L,), jnp.int32),
            pltpu.VMEM((256 * L,), jnp.int32),
            pltpu.VMEM((L, CH), jnp.float32),
            pltpu.VMEM((L, CH), jnp.float32),
            pltpu.VMEM((L * L,), jnp.float32),
            pltpu.VMEM((CAP * L,), jnp.float32),
            pltpu.VMEM((CAP * L,), jnp.int32),
            pltpu.VMEM((TOPK * L,), jnp.float32),
            pltpu.VMEM((TOPK * L,), jnp.int32),
            pltpu.VMEM((D_SPARSE,), jnp.float32),
            pltpu.SemaphoreType.DMA,
            pltpu.SemaphoreType.DMA,
        ],
    )(f3)


def kernel(x, W_enc, b_enc, W_dec, b_dec):
    x_cent = x - b_dec
    f = _encode(x_cent, W_enc, b_enc)
    f3 = f.reshape(GROUPS, L, D_SPARSE)
    z = _sc_topk_z(f3).reshape(N_TOKENS, D_SPARSE)
    x_hat = _decode(z, W_dec, b_dec)
    return (x_hat, z)
